# Initial kernel scaffold; baseline (speedup 1.0000x reference)
import jax
import jax.numpy as jnp
from jax.experimental import pallas as pl


def kernel(x):
    raise NotImplementedError("write your pallas kernel here")


# speedup vs baseline: 1.5331x; 1.5331x over previous
import functools

import jax
import jax.numpy as jnp
from jax import lax
from jax.experimental import pallas as pl
from jax.experimental.pallas import tpu as pltpu
from jax.experimental.pallas import tpu_sc as plsc

_R = 16384
_F = 26
_CARD = 64
_N = _R * _F
_NW = 32
_RPW = _N // _NW
_CHUNK = 512
_NCHUNK = _RPW // _CHUNK
_L = 16


def _body(x_hbm, out_hbm, idx_v, buf_v):
    wid = lax.axis_index("s") * 2 + lax.axis_index("c")
    base = wid * _RPW
    pltpu.sync_copy(x_hbm.at[pl.ds(base, _RPW)], idx_v)

    iota = lax.iota(jnp.int32, _L)
    ones = jnp.full((_L,), 1, jnp.int32)
    zeros = jnp.zeros((_L,), jnp.int32)

    def zinit(i, c):
        buf_v[pl.ds(i * _L, _L)] = zeros
        return c

    lax.fori_loop(0, _CHUNK * _CARD // _L, zinit, 0)

    def chunk(ci, c):
        r0 = ci * _CHUNK

        def fire(i, c2):
            vals = idx_v[pl.ds(r0 + i * _L, _L)]
            offs = (iota + i * _L) * _CARD + vals
            plsc.store_scatter(buf_v, [offs], ones)
            return c2

        lax.fori_loop(0, _CHUNK // _L, fire, 0)

        pltpu.sync_copy(
            buf_v, out_hbm.at[pl.ds((base + r0) * _CARD, _CHUNK * _CARD)]
        )

        def clear(i, c2):
            vals = idx_v[pl.ds(r0 + i * _L, _L)]
            offs = (iota + i * _L) * _CARD + vals
            plsc.store_scatter(buf_v, [offs], zeros)
            return c2

        lax.fori_loop(0, _CHUNK // _L, clear, 0)
        return c

    lax.fori_loop(0, _NCHUNK, chunk, 0)


@jax.jit
def _onehot(x_flat):
    mesh = plsc.VectorSubcoreMesh(core_axis_name="c", subcore_axis_name="s")
    f = functools.partial(
        pl.kernel,
        mesh=mesh,
        out_type=jax.ShapeDtypeStruct((_N * _CARD,), jnp.int32),
        scratch_types=[
            pltpu.VMEM((_RPW,), jnp.int32),
            pltpu.VMEM((_CHUNK * _CARD,), jnp.int32),
        ],
        compiler_params=pltpu.CompilerParams(needs_layout_passes=False),
    )(_body)
    return f(x_flat)


def kernel(x):
    out_flat = _onehot(x.reshape(-1).astype(jnp.int32))
    return out_flat.reshape(_R, _F * _CARD)


# speedup vs baseline: 1.6362x; 1.0672x over previous
import functools

import jax
import jax.numpy as jnp
from jax import lax
from jax.experimental import pallas as pl
from jax.experimental.pallas import tpu as pltpu
from jax.experimental.pallas import tpu_sc as plsc

_R = 16384
_F = 26
_CARD = 64
_N = _R * _F
_NW = 32
_RPW = _N // _NW
_CHUNK = 512
_NCHUNK = _RPW // _CHUNK
_L = 16


def _body(x_hbm, out_hbm, idx_v, buf0, buf1, sem0, sem1):
    wid = lax.axis_index("s") * 2 + lax.axis_index("c")
    base = wid * _RPW
    pltpu.sync_copy(x_hbm.at[pl.ds(base, _RPW)], idx_v)

    iota = lax.iota(jnp.int32, _L)
    ones = jnp.full((_L,), 1, jnp.int32)
    zeros = jnp.zeros((_L,), jnp.int32)
    bufs = (buf0, buf1)
    sems = (sem0, sem1)

    def zinit(i, c):
        buf0[pl.ds(i * _L, _L)] = zeros
        buf1[pl.ds(i * _L, _L)] = zeros
        return c

    lax.fori_loop(0, _CHUNK * _CARD // _L, zinit, 0)

    def poke(ci, b, val):
        r0 = ci * _CHUNK

        def step(i, c):
            vals = idx_v[pl.ds(r0 + i * _L, _L)]
            offs = (iota + i * _L) * _CARD + vals
            plsc.store_scatter(bufs[b], [offs], val)
            return c

        lax.fori_loop(0, _CHUNK // _L, step, 0)

    def start(ci, b):
        pltpu.make_async_copy(
            bufs[b],
            out_hbm.at[pl.ds((base + ci * _CHUNK) * _CARD, _CHUNK * _CARD)],
            sems[b],
        ).start()

    def drain(b):
        pltpu.make_async_copy(
            bufs[b],
            out_hbm.at[pl.ds(base * _CARD, _CHUNK * _CARD)],
            sems[b],
        ).wait()

    for b in (0, 1):
        poke(b, b, ones)
        start(b, b)

    def pair(p, c):
        for b in (0, 1):
            ci = 2 * p + b
            drain(b)
            poke(ci - 2, b, zeros)
            poke(ci, b, ones)
            start(ci, b)
        return c

    lax.fori_loop(1, _NCHUNK // 2, pair, 0)
    drain(0)
    drain(1)


@jax.jit
def _onehot(x_flat):
    mesh = plsc.VectorSubcoreMesh(core_axis_name="c", subcore_axis_name="s")
    f = functools.partial(
        pl.kernel,
        mesh=mesh,
        out_type=jax.ShapeDtypeStruct((_N * _CARD,), jnp.int32),
        scratch_types=[
            pltpu.VMEM((_RPW,), jnp.int32),
            pltpu.VMEM((_CHUNK * _CARD,), jnp.int32),
            pltpu.VMEM((_CHUNK * _CARD,), jnp.int32),
            pltpu.SemaphoreType.DMA,
            pltpu.SemaphoreType.DMA,
        ],
        compiler_params=pltpu.CompilerParams(needs_layout_passes=False),
    )(_body)
    return f(x_flat)


def kernel(x):
    out_flat = _onehot(x.reshape(-1).astype(jnp.int32))
    return out_flat.reshape(_R, _F * _CARD)


# speedup vs baseline: 4.0101x; 2.4508x over previous
import functools

import jax
import jax.numpy as jnp
from jax import lax
from jax.experimental import pallas as pl
from jax.experimental.pallas import tpu as pltpu
from jax.experimental.pallas import tpu_sc as plsc

_R = 16384
_F = 26
_CARD = 64
_W = _F * _CARD
_NW = 32
_RPW = _R // _NW
_CROWS = 16
_NCHUNK = _RPW // _CROWS
_CVALS = _CROWS * _F
_L = 16


def _body(x_hbm, out_hbm, idx_v, buf0, buf1, sem0, sem1):
    wid = lax.axis_index("s") * 2 + lax.axis_index("c")
    row0 = wid * _RPW
    pltpu.sync_copy(x_hbm.at[pl.ds(row0 * _F, _RPW * _F)], idx_v)

    iota = lax.iota(jnp.int32, _L)
    ones = jnp.full((_L,), 1, jnp.int32)
    zeros = jnp.zeros((_L,), jnp.int32)
    bufs = (buf0, buf1)
    sems = (sem0, sem1)

    def zinit(i, c):
        def zrow(r, c2):
            buf0[r, pl.ds(i * _L, _L)] = zeros
            buf1[r, pl.ds(i * _L, _L)] = zeros
            return c2

        lax.fori_loop(0, _CROWS, zrow, 0)
        return c

    lax.fori_loop(0, _W // _L, zinit, 0)

    def poke(ci, b, val):
        v0 = ci * _CVALS

        def step(i, c):
            g = iota + i * _L
            r = g // _F
            f = g - r * _F
            vals = idx_v[pl.ds(v0 + i * _L, _L)]
            plsc.store_scatter(bufs[b], [r, f * _CARD + vals], val)
            return c

        lax.fori_loop(0, _CVALS // _L, step, 0)

    def start(ci, b):
        pltpu.make_async_copy(
            bufs[b],
            out_hbm.at[pl.ds(row0 + ci * _CROWS, _CROWS), :],
            sems[b],
        ).start()

    def drain(b):
        pltpu.make_async_copy(
            bufs[b],
            out_hbm.at[pl.ds(row0, _CROWS), :],
            sems[b],
        ).wait()

    for b in (0, 1):
        poke(b, b, ones)
        start(b, b)

    def pair(p, c):
        for b in (0, 1):
            ci = 2 * p + b
            drain(b)
            poke(ci - 2, b, zeros)
            poke(ci, b, ones)
            start(ci, b)
        return c

    lax.fori_loop(1, _NCHUNK // 2, pair, 0)
    drain(0)
    drain(1)


@jax.jit
def _onehot(x_flat):
    mesh = plsc.VectorSubcoreMesh(core_axis_name="c", subcore_axis_name="s")
    f = functools.partial(
        pl.kernel,
        mesh=mesh,
        out_type=jax.ShapeDtypeStruct((_R, _W), jnp.int32),
        scratch_types=[
            pltpu.VMEM((_RPW * _F,), jnp.int32),
            pltpu.VMEM((_CROWS, _W), jnp.int32),
            pltpu.VMEM((_CROWS, _W), jnp.int32),
            pltpu.SemaphoreType.DMA,
            pltpu.SemaphoreType.DMA,
        ],
        compiler_params=pltpu.CompilerParams(needs_layout_passes=False),
    )(_body)
    return f(x_flat)


def kernel(x):
    return _onehot(x.reshape(-1).astype(jnp.int32))


# speedup vs baseline: 4.8089x; 1.1992x over previous
import functools

import jax
import jax.numpy as jnp
from jax import lax
from jax.experimental import pallas as pl
from jax.experimental.pallas import tpu as pltpu
from jax.experimental.pallas import tpu_sc as plsc

_R = 16384
_F = 26
_CARD = 64
_W = _F * _CARD
_NW = 32
_RPW = _R // _NW
_CROWS = 16
_NCHUNK = _RPW // _CROWS
_CVALS = _CROWS * _F
_L = 16


def _body(x_hbm, out_hbm, idx_v, buf0, buf1, rtab, ctab, sem0, sem1):
    wid = lax.axis_index("s") * 2 + lax.axis_index("c")
    row0 = wid * _RPW
    pltpu.sync_copy(x_hbm.at[pl.ds(row0, _RPW), :], idx_v)

    iota = lax.iota(jnp.int32, _L)
    ones = jnp.full((_L,), 1, jnp.int32)
    zeros = jnp.zeros((_L,), jnp.int32)
    bufs = (buf0, buf1)
    sems = (sem0, sem1)

    def tinit(i, c):
        g = iota + i * _L
        r = g // _F
        f = g - r * _F
        rtab[pl.ds(i * _L, _L)] = r
        ctab[pl.ds(i * _L, _L)] = f

        def zrow(rr, c2):
            buf0[rr, pl.ds(i * _L * 4, _L)] = zeros
            buf1[rr, pl.ds(i * _L * 4, _L)] = zeros
            buf0[rr, pl.ds(i * _L * 4 + _L, _L)] = zeros
            buf1[rr, pl.ds(i * _L * 4 + _L, _L)] = zeros
            buf0[rr, pl.ds(i * _L * 4 + 2 * _L, _L)] = zeros
            buf1[rr, pl.ds(i * _L * 4 + 2 * _L, _L)] = zeros
            buf0[rr, pl.ds(i * _L * 4 + 3 * _L, _L)] = zeros
            buf1[rr, pl.ds(i * _L * 4 + 3 * _L, _L)] = zeros
            return c2

        lax.fori_loop(0, _CROWS, zrow, 0)
        return c

    lax.fori_loop(0, _CVALS // _L, tinit, 0)

    def poke(ci, b, val):
        rbase = ci * _CROWS

        def step(i, c):
            r = rtab[pl.ds(i * _L, _L)]
            f = ctab[pl.ds(i * _L, _L)]
            vals = plsc.load_gather(idx_v, [rbase + r, f])
            plsc.store_scatter(bufs[b], [r, f * _CARD + vals], val)
            return c

        lax.fori_loop(0, _CVALS // _L, step, 0)

    def start(ci, b):
        pltpu.make_async_copy(
            bufs[b],
            out_hbm.at[pl.ds(row0 + ci * _CROWS, _CROWS), :],
            sems[b],
        ).start()

    def drain(b):
        pltpu.make_async_copy(
            bufs[b],
            out_hbm.at[pl.ds(row0, _CROWS), :],
            sems[b],
        ).wait()

    for b in (0, 1):
        poke(b, b, ones)
        start(b, b)

    def pair(p, c):
        for b in (0, 1):
            ci = 2 * p + b
            drain(b)
            poke(ci - 2, b, zeros)
            poke(ci, b, ones)
            start(ci, b)
        return c

    lax.fori_loop(1, _NCHUNK // 2, pair, 0)
    drain(0)
    drain(1)


@jax.jit
def _onehot(x):
    mesh = plsc.VectorSubcoreMesh(core_axis_name="c", subcore_axis_name="s")
    f = functools.partial(
        pl.kernel,
        mesh=mesh,
        out_type=jax.ShapeDtypeStruct((_R, _W), jnp.int32),
        scratch_types=[
            pltpu.VMEM((_RPW, _F), jnp.int32),
            pltpu.VMEM((_CROWS, _W), jnp.int32),
            pltpu.VMEM((_CROWS, _W), jnp.int32),
            pltpu.VMEM((_CVALS,), jnp.int32),
            pltpu.VMEM((_CVALS,), jnp.int32),
            pltpu.SemaphoreType.DMA,
            pltpu.SemaphoreType.DMA,
        ],
        compiler_params=pltpu.CompilerParams(needs_layout_passes=False),
    )(_body)
    return f(x)


def kernel(x):
    return _onehot(x.astype(jnp.int32))
